# baseline (speedup 1.0000x reference)
import functools

import jax
import jax.numpy as jnp
from jax.experimental import pallas as pl

S, D = 2048, 2048
H, KV, HD = 16, 4, 128
EPS = 1e-6
SCALE = HD ** -0.5

BQ = 512
BK = 512
GRP = 4


def _rot_half(x):
    half = x.shape[-1] // 2
    return jnp.concatenate([-x[:, half:], x[:, :half]], axis=-1)


def _norm_rope(x, w, c, s):
    var = jnp.mean(x * x, axis=-1, keepdims=True)
    x = x * jax.lax.rsqrt(var + EPS) * w
    return x * c + _rot_half(x) * s


def _qproj_kernel(x_ref, wq_ref, cos_ref, sin_ref, qw_ref, q_ref, g_ref):
    acc = jnp.dot(x_ref[...], wq_ref[...], preferred_element_type=jnp.float32)
    c = cos_ref[...]
    s = sin_ref[...]
    qw = qw_ref[...]
    gs = []
    for hh in range(2):
        qh = acc[:, hh * 256:hh * 256 + HD]
        gh = acc[:, hh * 256 + HD:hh * 256 + 2 * HD]
        q_ref[hh, :, :] = (_norm_rope(qh, qw, c, s) * SCALE).astype(jnp.bfloat16)
        gs.append(gh)
    g_ref[...] = jnp.concatenate(gs, axis=1)


def _kvproj_kernel(x_ref, wk_ref, wv_ref, cos_ref, sin_ref, kw_ref, k_ref, v_ref):
    xb = x_ref[...]
    kacc = jnp.dot(xb, wk_ref[...], preferred_element_type=jnp.float32)
    v_ref[...] = jnp.dot(xb, wv_ref[...], preferred_element_type=jnp.float32).astype(jnp.bfloat16)
    c = cos_ref[...]
    s = sin_ref[...]
    kw = kw_ref[...]
    ks = [_norm_rope(kacc[:, h * HD:(h + 1) * HD], kw, c, s) for h in range(KV)]
    k_ref[...] = jnp.concatenate(ks, axis=1).astype(jnp.bfloat16)


def _attn_kernel(q_ref, k_ref, v_ref, g_ref, o_ref):
    i = pl.program_id(1)
    qb = q_ref[...].reshape(GRP * BQ, HD)

    def blk(j, l, acc, masked):
        kb = k_ref[pl.ds(j * BK, BK), :]
        sc = jax.lax.dot_general(qb, kb, (((1,), (1,)), ((), ())),
                                 preferred_element_type=jnp.float32)
        p = jnp.exp(sc)
        if masked:
            rows = jax.lax.broadcasted_iota(jnp.int32, (GRP, BQ, BK), 1)
            cols = jax.lax.broadcasted_iota(jnp.int32, (GRP, BQ, BK), 2)
            tri = (cols <= rows).reshape(GRP * BQ, BK)
            p = jnp.where(tri, p, 0.0)
        l = l + jnp.sum(p, axis=1, keepdims=True)
        vb = v_ref[pl.ds(j * BK, BK), :]
        acc = acc + jnp.dot(p.astype(jnp.bfloat16), vb,
                            preferred_element_type=jnp.float32)
        return l, acc

    l0 = jnp.zeros((GRP * BQ, 1), jnp.float32)
    a0 = jnp.zeros((GRP * BQ, HD), jnp.float32)

    def make_branch(n):
        def f():
            l, acc = l0, a0
            for j in range(n):
                l, acc = blk(j, l, acc, masked=False)
            return l, acc
        return f

    l, acc = jax.lax.switch(i, [make_branch(n) for n in range(S // BQ)])
    l, acc = blk(i, l, acc, masked=True)

    o = acc / l
    o = jnp.concatenate([o[hh * BQ:(hh + 1) * BQ, :] for hh in range(GRP)],
                        axis=1)
    o = o * jax.nn.sigmoid(g_ref[...])
    o_ref[...] = o.astype(jnp.bfloat16)


def _oproj_kernel(a_ref, wo_ref, y_ref):
    y_ref[...] = jnp.dot(a_ref[...], wo_ref[...], preferred_element_type=jnp.float32)


@functools.partial(jax.jit, static_argnums=())
def kernel(hidden_states, cos, sin, Wq, Wk, Wv, Wo, q_norm_w, k_norm_w):
    x = hidden_states[0].astype(jnp.bfloat16)
    Wq = Wq.astype(jnp.bfloat16)
    Wk = Wk.astype(jnp.bfloat16)
    Wv = Wv.astype(jnp.bfloat16)
    Wo = Wo.astype(jnp.bfloat16)
    c2 = cos[0]
    s2 = sin[0]
    qw = q_norm_w.reshape(1, HD)
    kw = k_norm_w.reshape(1, HD)

    q, gate = pl.pallas_call(
        _qproj_kernel,
        grid=(H // 2,),
        in_specs=[
            pl.BlockSpec((S, D), lambda n: (0, 0)),
            pl.BlockSpec((D, 512), lambda n: (0, n)),
            pl.BlockSpec((S, HD), lambda n: (0, 0)),
            pl.BlockSpec((S, HD), lambda n: (0, 0)),
            pl.BlockSpec((1, HD), lambda n: (0, 0)),
        ],
        out_specs=[
            pl.BlockSpec((2, S, HD), lambda n: (n, 0, 0)),
            pl.BlockSpec((S, 2 * HD), lambda n: (0, n)),
        ],
        out_shape=[
            jax.ShapeDtypeStruct((H, S, HD), jnp.bfloat16),
            jax.ShapeDtypeStruct((S, H * HD), jnp.float32),
        ],
    )(x, Wq, c2, s2, qw)

    k, v = pl.pallas_call(
        _kvproj_kernel,
        grid=(1,),
        in_specs=[
            pl.BlockSpec((S, D), lambda i: (0, 0)),
            pl.BlockSpec((D, KV * HD), lambda i: (0, 0)),
            pl.BlockSpec((D, KV * HD), lambda i: (0, 0)),
            pl.BlockSpec((S, HD), lambda i: (0, 0)),
            pl.BlockSpec((S, HD), lambda i: (0, 0)),
            pl.BlockSpec((1, HD), lambda i: (0, 0)),
        ],
        out_specs=[
            pl.BlockSpec((S, KV * HD), lambda i: (0, 0)),
            pl.BlockSpec((S, KV * HD), lambda i: (0, 0)),
        ],
        out_shape=[
            jax.ShapeDtypeStruct((S, KV * HD), jnp.bfloat16),
            jax.ShapeDtypeStruct((S, KV * HD), jnp.bfloat16),
        ],
    )(x, Wk, Wv, c2, s2, kw)

    attn = pl.pallas_call(
        _attn_kernel,
        grid=(KV, S // BQ),
        in_specs=[
            pl.BlockSpec((GRP, BQ, HD), lambda g, i: (g, i, 0)),
            pl.BlockSpec((S, HD), lambda g, i: (0, g)),
            pl.BlockSpec((S, HD), lambda g, i: (0, g)),
            pl.BlockSpec((BQ, GRP * HD), lambda g, i: (i, g)),
        ],
        out_specs=pl.BlockSpec((BQ, GRP * HD), lambda g, i: (i, g)),
        out_shape=jax.ShapeDtypeStruct((S, H * HD), jnp.bfloat16),
    )(q, k, v, gate)

    y = pl.pallas_call(
        _oproj_kernel,
        grid=(D // 512,),
        in_specs=[
            pl.BlockSpec((S, H * HD), lambda n: (0, 0)),
            pl.BlockSpec((H * HD, 512), lambda n: (0, n)),
        ],
        out_specs=pl.BlockSpec((S, 512), lambda n: (0, n)),
        out_shape=jax.ShapeDtypeStruct((S, D), jnp.float32),
    )(attn, Wo)

    return y[None]


# speedup vs baseline: 1.0659x; 1.0659x over previous
import functools

import jax
import jax.numpy as jnp
from jax.experimental import pallas as pl

S, D = 2048, 2048
H, KV, HD = 16, 4, 128
EPS = 1e-6
SCALE = HD ** -0.5

BQ = 512
BK = 512
GRP = 4


def _rot_half(x):
    half = x.shape[-1] // 2
    return jnp.concatenate([-x[:, half:], x[:, :half]], axis=-1)


def _norm_rope(x, w, c, s):
    var = jnp.mean(x * x, axis=-1, keepdims=True)
    x = x * jax.lax.rsqrt(var + EPS) * w
    return x * c + _rot_half(x) * s


def _qproj_kernel(x_ref, wq_ref, cos_ref, sin_ref, qw_ref, q_ref, g_ref):
    acc = jnp.dot(x_ref[...], wq_ref[...], preferred_element_type=jnp.float32)
    c = cos_ref[...]
    s = sin_ref[...]
    qw = qw_ref[...]
    gs = []
    for hh in range(2):
        qh = acc[:, hh * 256:hh * 256 + HD]
        gh = acc[:, hh * 256 + HD:hh * 256 + 2 * HD]
        q_ref[hh, :, :] = (_norm_rope(qh, qw, c, s) * SCALE).astype(jnp.bfloat16)
        gs.append(gh)
    g_ref[...] = jnp.concatenate(gs, axis=1)


def _kvproj_kernel(x_ref, wk_ref, wv_ref, cos_ref, sin_ref, kw_ref, k_ref, v_ref):
    xb = x_ref[...]
    kacc = jnp.dot(xb, wk_ref[...], preferred_element_type=jnp.float32)
    v_ref[...] = jnp.dot(xb, wv_ref[...], preferred_element_type=jnp.float32).astype(jnp.bfloat16)
    c = cos_ref[...]
    s = sin_ref[...]
    kw = kw_ref[...]
    ks = [_norm_rope(kacc[:, h * HD:(h + 1) * HD], kw, c, s) for h in range(KV)]
    k_ref[...] = jnp.concatenate(ks, axis=1).astype(jnp.bfloat16)


def _attn_kernel(q_ref, k_ref, v_ref, g_ref, o_ref):
    i = pl.program_id(1)
    qb = q_ref[...].reshape(GRP * BQ, HD)

    def blk(j, l, acc, masked):
        kb = k_ref[pl.ds(j * BK, BK), :]
        sc = jax.lax.dot_general(qb, kb, (((1,), (1,)), ((), ())),
                                 preferred_element_type=jnp.float32)
        p = jnp.exp(sc)
        if masked:
            rows = jax.lax.broadcasted_iota(jnp.int32, (GRP, BQ, BK), 1)
            cols = jax.lax.broadcasted_iota(jnp.int32, (GRP, BQ, BK), 2)
            tri = (cols <= rows).reshape(GRP * BQ, BK)
            p = jnp.where(tri, p, 0.0)
        l = l + jnp.sum(p, axis=1, keepdims=True)
        vb = v_ref[pl.ds(j * BK, BK), :]
        acc = acc + jnp.dot(p.astype(jnp.bfloat16), vb,
                            preferred_element_type=jnp.float32)
        return l, acc

    def body(j, carry):
        l, acc = carry
        return blk(j, l, acc, masked=False)

    l0 = jnp.zeros((GRP * BQ, 1), jnp.float32)
    a0 = jnp.zeros((GRP * BQ, HD), jnp.float32)
    l, acc = jax.lax.fori_loop(0, i, body, (l0, a0))
    l, acc = blk(i, l, acc, masked=True)

    o = acc / l
    o = jnp.concatenate([o[hh * BQ:(hh + 1) * BQ, :] for hh in range(GRP)],
                        axis=1)
    o = o * jax.nn.sigmoid(g_ref[...])
    o_ref[...] = o.astype(jnp.bfloat16)


def _oproj_kernel(a_ref, wo_ref, y_ref):
    y_ref[...] = jnp.dot(a_ref[...], wo_ref[...], preferred_element_type=jnp.float32)


@functools.partial(jax.jit, static_argnums=())
def kernel(hidden_states, cos, sin, Wq, Wk, Wv, Wo, q_norm_w, k_norm_w):
    x = hidden_states[0].astype(jnp.bfloat16)
    Wq = Wq.astype(jnp.bfloat16)
    Wk = Wk.astype(jnp.bfloat16)
    Wv = Wv.astype(jnp.bfloat16)
    Wo = Wo.astype(jnp.bfloat16)
    c2 = cos[0]
    s2 = sin[0]
    qw = q_norm_w.reshape(1, HD)
    kw = k_norm_w.reshape(1, HD)

    q, gate = pl.pallas_call(
        _qproj_kernel,
        grid=(H // 2,),
        in_specs=[
            pl.BlockSpec((S, D), lambda n: (0, 0)),
            pl.BlockSpec((D, 512), lambda n: (0, n)),
            pl.BlockSpec((S, HD), lambda n: (0, 0)),
            pl.BlockSpec((S, HD), lambda n: (0, 0)),
            pl.BlockSpec((1, HD), lambda n: (0, 0)),
        ],
        out_specs=[
            pl.BlockSpec((2, S, HD), lambda n: (n, 0, 0)),
            pl.BlockSpec((S, 2 * HD), lambda n: (0, n)),
        ],
        out_shape=[
            jax.ShapeDtypeStruct((H, S, HD), jnp.bfloat16),
            jax.ShapeDtypeStruct((S, H * HD), jnp.float32),
        ],
    )(x, Wq, c2, s2, qw)

    k, v = pl.pallas_call(
        _kvproj_kernel,
        grid=(1,),
        in_specs=[
            pl.BlockSpec((S, D), lambda i: (0, 0)),
            pl.BlockSpec((D, KV * HD), lambda i: (0, 0)),
            pl.BlockSpec((D, KV * HD), lambda i: (0, 0)),
            pl.BlockSpec((S, HD), lambda i: (0, 0)),
            pl.BlockSpec((S, HD), lambda i: (0, 0)),
            pl.BlockSpec((1, HD), lambda i: (0, 0)),
        ],
        out_specs=[
            pl.BlockSpec((S, KV * HD), lambda i: (0, 0)),
            pl.BlockSpec((S, KV * HD), lambda i: (0, 0)),
        ],
        out_shape=[
            jax.ShapeDtypeStruct((S, KV * HD), jnp.bfloat16),
            jax.ShapeDtypeStruct((S, KV * HD), jnp.bfloat16),
        ],
    )(x, Wk, Wv, c2, s2, kw)

    attn = pl.pallas_call(
        _attn_kernel,
        grid=(KV, S // BQ),
        in_specs=[
            pl.BlockSpec((GRP, BQ, HD), lambda g, i: (g, i, 0)),
            pl.BlockSpec((S, HD), lambda g, i: (0, g)),
            pl.BlockSpec((S, HD), lambda g, i: (0, g)),
            pl.BlockSpec((BQ, GRP * HD), lambda g, i: (i, g)),
        ],
        out_specs=pl.BlockSpec((BQ, GRP * HD), lambda g, i: (i, g)),
        out_shape=jax.ShapeDtypeStruct((S, H * HD), jnp.bfloat16),
    )(q, k, v, gate)

    y = pl.pallas_call(
        _oproj_kernel,
        grid=(D // 512,),
        in_specs=[
            pl.BlockSpec((S, H * HD), lambda n: (0, 0)),
            pl.BlockSpec((H * HD, 512), lambda n: (0, n)),
        ],
        out_specs=pl.BlockSpec((S, 512), lambda n: (0, n)),
        out_shape=jax.ShapeDtypeStruct((S, D), jnp.float32),
    )(attn, Wo)

    return y[None]


# speedup vs baseline: 1.1412x; 1.0706x over previous
import functools

import jax
import jax.numpy as jnp
from jax.experimental import pallas as pl

S, D = 2048, 2048
H, KV, HD = 16, 4, 128
EPS = 1e-6
SCALE = HD ** -0.5

BQ = 512
BK = 512
GRP = 4


def _rot_half(x):
    half = x.shape[-1] // 2
    return jnp.concatenate([-x[:, half:], x[:, :half]], axis=-1)


def _norm_rope(x, w, c, s):
    var = jnp.mean(x * x, axis=-1, keepdims=True)
    x = x * jax.lax.rsqrt(var + EPS) * w
    return x * c + _rot_half(x) * s


def _qproj_kernel(x_ref, wq_ref, cos_ref, sin_ref, qw_ref, q_ref, g_ref):
    acc = jnp.dot(x_ref[...], wq_ref[...], preferred_element_type=jnp.float32)
    c = cos_ref[...]
    s = sin_ref[...]
    qw = qw_ref[...]
    gs = []
    for hh in range(4):
        qh = acc[:, hh * 256:hh * 256 + HD]
        gh = acc[:, hh * 256 + HD:hh * 256 + 2 * HD]
        q_ref[hh, :, :] = (_norm_rope(qh, qw, c, s) * SCALE).astype(jnp.bfloat16)
        gs.append(gh)
    g_ref[...] = jnp.concatenate(gs, axis=1)


def _kvproj_kernel(x_ref, wk_ref, wv_ref, cos_ref, sin_ref, kw_ref, k_ref, v_ref):
    xb = x_ref[...]
    kacc = jnp.dot(xb, wk_ref[...], preferred_element_type=jnp.float32)
    v_ref[...] = jnp.dot(xb, wv_ref[...], preferred_element_type=jnp.float32).astype(jnp.bfloat16)
    c = cos_ref[...]
    s = sin_ref[...]
    kw = kw_ref[...]
    ks = [_norm_rope(kacc[:, h * HD:(h + 1) * HD], kw, c, s) for h in range(KV)]
    k_ref[...] = jnp.concatenate(ks, axis=1).astype(jnp.bfloat16)


def _attn_kernel(q_ref, k_ref, v_ref, g_ref, o_ref):
    i = pl.program_id(1)
    qb = q_ref[...].reshape(GRP * BQ, HD)

    def blk(j, l, acc, masked):
        kb = k_ref[pl.ds(j * BK, BK), :]
        sc = jax.lax.dot_general(qb, kb, (((1,), (1,)), ((), ())),
                                 preferred_element_type=jnp.float32)
        p = jnp.exp(sc)
        if masked:
            rows = jax.lax.broadcasted_iota(jnp.int32, (GRP, BQ, BK), 1)
            cols = jax.lax.broadcasted_iota(jnp.int32, (GRP, BQ, BK), 2)
            tri = (cols <= rows).reshape(GRP * BQ, BK)
            p = jnp.where(tri, p, 0.0)
        l = l + jnp.sum(p, axis=1, keepdims=True)
        vb = v_ref[pl.ds(j * BK, BK), :]
        acc = acc + jnp.dot(p.astype(jnp.bfloat16), vb,
                            preferred_element_type=jnp.float32)
        return l, acc

    def body(j, carry):
        l, acc = carry
        return blk(j, l, acc, masked=False)

    l0 = jnp.zeros((GRP * BQ, 1), jnp.float32)
    a0 = jnp.zeros((GRP * BQ, HD), jnp.float32)
    l, acc = jax.lax.fori_loop(0, i, body, (l0, a0))
    l, acc = blk(i, l, acc, masked=True)

    o = acc / l
    o = jnp.concatenate([o[hh * BQ:(hh + 1) * BQ, :] for hh in range(GRP)],
                        axis=1)
    o = o * jax.nn.sigmoid(g_ref[...])
    o_ref[...] = o.astype(jnp.bfloat16)


def _oproj_kernel(a_ref, wo_ref, y_ref):
    y_ref[...] = jnp.dot(a_ref[...], wo_ref[...], preferred_element_type=jnp.float32)


@functools.partial(jax.jit, static_argnums=())
def kernel(hidden_states, cos, sin, Wq, Wk, Wv, Wo, q_norm_w, k_norm_w):
    x = hidden_states[0].astype(jnp.bfloat16)
    Wq = Wq.astype(jnp.bfloat16)
    Wk = Wk.astype(jnp.bfloat16)
    Wv = Wv.astype(jnp.bfloat16)
    Wo = Wo.astype(jnp.bfloat16)
    c2 = cos[0]
    s2 = sin[0]
    qw = q_norm_w.reshape(1, HD)
    kw = k_norm_w.reshape(1, HD)

    q, gate = pl.pallas_call(
        _qproj_kernel,
        grid=(H // 4,),
        in_specs=[
            pl.BlockSpec((S, D), lambda n: (0, 0)),
            pl.BlockSpec((D, 1024), lambda n: (0, n)),
            pl.BlockSpec((S, HD), lambda n: (0, 0)),
            pl.BlockSpec((S, HD), lambda n: (0, 0)),
            pl.BlockSpec((1, HD), lambda n: (0, 0)),
        ],
        out_specs=[
            pl.BlockSpec((4, S, HD), lambda n: (n, 0, 0)),
            pl.BlockSpec((S, 4 * HD), lambda n: (0, n)),
        ],
        out_shape=[
            jax.ShapeDtypeStruct((H, S, HD), jnp.bfloat16),
            jax.ShapeDtypeStruct((S, H * HD), jnp.float32),
        ],
    )(x, Wq, c2, s2, qw)

    k, v = pl.pallas_call(
        _kvproj_kernel,
        grid=(1,),
        in_specs=[
            pl.BlockSpec((S, D), lambda i: (0, 0)),
            pl.BlockSpec((D, KV * HD), lambda i: (0, 0)),
            pl.BlockSpec((D, KV * HD), lambda i: (0, 0)),
            pl.BlockSpec((S, HD), lambda i: (0, 0)),
            pl.BlockSpec((S, HD), lambda i: (0, 0)),
            pl.BlockSpec((1, HD), lambda i: (0, 0)),
        ],
        out_specs=[
            pl.BlockSpec((S, KV * HD), lambda i: (0, 0)),
            pl.BlockSpec((S, KV * HD), lambda i: (0, 0)),
        ],
        out_shape=[
            jax.ShapeDtypeStruct((S, KV * HD), jnp.bfloat16),
            jax.ShapeDtypeStruct((S, KV * HD), jnp.bfloat16),
        ],
    )(x, Wk, Wv, c2, s2, kw)

    attn = pl.pallas_call(
        _attn_kernel,
        grid=(KV, S // BQ),
        in_specs=[
            pl.BlockSpec((GRP, BQ, HD), lambda g, i: (g, i, 0)),
            pl.BlockSpec((S, HD), lambda g, i: (0, g)),
            pl.BlockSpec((S, HD), lambda g, i: (0, g)),
            pl.BlockSpec((BQ, GRP * HD), lambda g, i: (i, g)),
        ],
        out_specs=pl.BlockSpec((BQ, GRP * HD), lambda g, i: (i, g)),
        out_shape=jax.ShapeDtypeStruct((S, H * HD), jnp.bfloat16),
    )(q, k, v, gate)

    y = pl.pallas_call(
        _oproj_kernel,
        grid=(D // 512,),
        in_specs=[
            pl.BlockSpec((S, H * HD), lambda n: (0, 0)),
            pl.BlockSpec((H * HD, 512), lambda n: (0, n)),
        ],
        out_specs=pl.BlockSpec((S, 512), lambda n: (0, n)),
        out_shape=jax.ShapeDtypeStruct((S, D), jnp.float32),
    )(attn, Wo)

    return y[None]
